# baseline (speedup 1.0000x reference)
import functools

import jax
import jax.numpy as jnp
from jax import lax
from jax.experimental import pallas as pl
from jax.experimental.pallas import tpu as pltpu
from jax.experimental.pallas import tpu_sc as plsc

NUM_CORES = 2
NUM_SUBCORES = 16
NUM_WORKERS = NUM_CORES * NUM_SUBCORES
LANES = 16

BATCH = 16384
DIM = 32
B_PER_W = BATCH // NUM_WORKERS
CHUNK = 128
N_CHUNKS = B_PER_W // CHUNK


def _gmf_body(uidx_hbm, iidx_hbm, utab_hbm, itab_hbm, params_hbm, out_hbm,
              uidx_v, iidx_v, u_rows, i_rows, qT, out_v, params_v,
              sem0, sem1, sem2, sem3):
    sems = (sem0, sem1, sem2, sem3)
    wid = lax.axis_index("s") * NUM_CORES + lax.axis_index("c")
    base = wid * B_PER_W
    crow = wid * N_CHUNKS

    pltpu.sync_copy(uidx_hbm.at[pl.ds(crow, N_CHUNKS)], uidx_v)
    pltpu.sync_copy(iidx_hbm.at[pl.ds(crow, N_CHUNKS)], iidx_v)
    pltpu.sync_copy(params_hbm, params_v)

    copies = []
    for c in range(N_CHUNKS):
        cu = pltpu.async_copy(utab_hbm.at[uidx_v.at[c]],
                              u_rows.at[pl.ds(c * CHUNK, CHUNK)], sems[c])
        ci = pltpu.async_copy(itab_hbm.at[iidx_v.at[c]],
                              i_rows.at[pl.ds(c * CHUNK, CHUNK)], sems[c])
        copies.append((cu, ci))

    w0 = params_v[pl.ds(0, 16)]
    w1 = params_v[pl.ds(16, 16)]
    bv = params_v[pl.ds(32, 16)]
    iota = lax.iota(jnp.int32, LANES)

    for c in range(N_CHUNKS):
        cu, ci = copies[c]
        cu.wait()
        ci.wait()
        row0 = c * CHUNK

        def row_body(r, carry):
            u0 = u_rows[r, pl.ds(0, 16)]
            u1 = u_rows[r, pl.ds(16, 16)]
            i0 = i_rows[r, pl.ds(0, 16)]
            i1 = i_rows[r, pl.ds(16, 16)]
            p = u0 * i0 * w0 + u1 * i1 * w1
            plsc.store_scatter(qT, [iota, jnp.full((LANES,), r, jnp.int32)], p)
            return carry

        lax.fori_loop(row0, row0 + CHUNK, row_body, 0, unroll=4)

        def grp_body(g, carry):
            col = g * LANES
            acc = bv
            for j in range(LANES):
                acc = acc + qT[j, pl.ds(col, LANES)]
            out_v[pl.ds(col, LANES)] = 1.0 / (1.0 + jnp.exp(-acc))
            return carry

        lax.fori_loop(row0 // LANES, (row0 + CHUNK) // LANES, grp_body, 0)

    pltpu.sync_copy(out_v, out_hbm.at[pl.ds(base, B_PER_W)])


@jax.jit
def _gmf(uidx, iidx, utab, itab, params):
    mesh = plsc.VectorSubcoreMesh(core_axis_name="c", subcore_axis_name="s",
                                  num_cores=NUM_CORES,
                                  num_subcores=NUM_SUBCORES)
    fn = pl.kernel(
        _gmf_body,
        out_type=jax.ShapeDtypeStruct((BATCH,), jnp.float32),
        mesh=mesh,
        compiler_params=pltpu.CompilerParams(use_tc_tiling_on_sc=False,
                                             needs_layout_passes=False),
        scratch_types=[
            pltpu.VMEM((N_CHUNKS, CHUNK), jnp.int32),
            pltpu.VMEM((N_CHUNKS, CHUNK), jnp.int32),
            pltpu.VMEM((B_PER_W, DIM), jnp.float32),
            pltpu.VMEM((B_PER_W, DIM), jnp.float32),
            pltpu.VMEM((LANES, B_PER_W), jnp.float32),
            pltpu.VMEM((B_PER_W,), jnp.float32),
            pltpu.VMEM((48,), jnp.float32),
            pltpu.SemaphoreType.DMA,
            pltpu.SemaphoreType.DMA,
            pltpu.SemaphoreType.DMA,
            pltpu.SemaphoreType.DMA,
        ],
    )
    return fn(uidx, iidx, utab, itab, params)


def kernel(user_indices, item_indices, user_table, item_table, W, b):
    uidx = user_indices.astype(jnp.int32).reshape(NUM_WORKERS * N_CHUNKS, CHUNK)
    iidx = item_indices.astype(jnp.int32).reshape(NUM_WORKERS * N_CHUNKS, CHUNK)
    params = jnp.concatenate(
        [W.reshape(-1).astype(jnp.float32),
         jnp.broadcast_to(b.astype(jnp.float32), (16,))])
    return _gmf(uidx, iidx, user_table, item_table, params)
